# Initial kernel scaffold; baseline (speedup 1.0000x reference)
import jax
import jax.numpy as jnp
from jax.experimental import pallas as pl


def kernel(nh, eh, edge_index, nf_W1, nf_b1, nf_W2, nf_b2, nf_eps, nf_gamma, nf_beta, ef_W1, ef_b1, ef_W2, ef_b2, ef_eps, ef_gamma, ef_beta):
    raise NotImplementedError("write your pallas kernel here")


# speedup vs baseline: 3.6719x; 3.6719x over previous
import functools
import jax
import jax.numpy as jnp
from jax import lax
from jax.experimental import pallas as pl
from jax.experimental.pallas import tpu as pltpu
from jax.experimental.pallas import tpu_sc as plsc

N = 10000
E = 320000
D = 128
NPAD = 10240
NW = 32
EPW = 10240
C = 128
NCHUNK = EPW // C
NEG = -3.0e38

_mesh = plsc.VectorSubcoreMesh(core_axis_name="c", subcore_axis_name="s")


def _wid():
    return lax.axis_index("s") * 2 + lax.axis_index("c")


def _fill(ref, n, val):
    def body(i, _):
        ref[pl.ds(i * 16, 16)] = jnp.full((16,), val, jnp.float32)
        return 0
    lax.fori_loop(0, n // 16, body, 0)


@functools.partial(
    pl.kernel, mesh=_mesh,
    compiler_params=pltpu.CompilerParams(needs_layout_passes=False),
    out_type=(jax.ShapeDtypeStruct((E,), jnp.float32),
              jax.ShapeDtypeStruct((NW, NPAD), jnp.float32)),
    scratch_types=(pltpu.VMEM((C,), jnp.int32),
                   pltpu.VMEM((C,), jnp.int32),
                   pltpu.VMEM((C, D), jnp.float32),
                   pltpu.VMEM((C, D), jnp.float32),
                   pltpu.VMEM((C, D), jnp.float32),
                   pltpu.VMEM((C,), jnp.float32),
                   pltpu.VMEM((NPAD + 16,), jnp.float32),
                   pltpu.SemaphoreType.DMA),
)
def _s1(nh, eh, src, dst, attn_out, maxpart_out,
        idx_s, idx_d, srows, drows, erows, attn_buf, maxloc, sem):
    w = _wid()
    _fill(maxloc, NPAD + 16, NEG)
    lane = lax.iota(jnp.int32, 16)
    mask0 = lane == 0

    def chunk(k, _):
        base = w * EPW + k * C

        @pl.when(base < E)
        def _():
            pltpu.sync_copy(src.at[pl.ds(base, C)], idx_s)
            pltpu.sync_copy(dst.at[pl.ds(base, C)], idx_d)
            c1 = pltpu.async_copy(nh.at[idx_s], srows, sem)
            c2 = pltpu.async_copy(nh.at[idx_d], drows, sem)
            c3 = pltpu.async_copy(eh.at[pl.ds(base, C)], erows, sem)
            c1.wait()
            c2.wait()
            c3.wait()

            def grp(gi, _):
                vec = jnp.zeros((16,), jnp.float32)
                dv = idx_d[pl.ds(gi * 16, 16)]
                for e in range(16):
                    i = gi * 16 + e
                    acc = jnp.zeros((16,), jnp.float32)
                    for j in range(8):
                        sl = pl.ds(j * 16, 16)
                        acc += (srows[i, sl] + erows[i, sl]) * drows[i, sl]
                    v = jnp.sum(acc)
                    vec = jnp.where(lane == e, v, vec)
                    d = dv[e]
                    old = maxloc[pl.ds(d, 16)]
                    maxloc[pl.ds(d, 16)] = jnp.where(
                        mask0, jnp.maximum(old, v), old)
                attn_buf[pl.ds(gi * 16, 16)] = vec
                return 0
            lax.fori_loop(0, C // 16, grp, 0)
            pltpu.sync_copy(attn_buf, attn_out.at[pl.ds(base, C)])
        return 0
    lax.fori_loop(0, NCHUNK, chunk, 0)
    pltpu.sync_copy(maxloc.at[pl.ds(0, NPAD)], maxpart_out.at[w])


@functools.partial(
    pl.kernel, mesh=_mesh,
    compiler_params=pltpu.CompilerParams(needs_layout_passes=False),
    out_type=(jax.ShapeDtypeStruct((E,), jnp.float32),
              jax.ShapeDtypeStruct((NW, NPAD), jnp.float32)),
    scratch_types=(pltpu.VMEM((NPAD,), jnp.float32),
                   pltpu.VMEM((NPAD,), jnp.float32),
                   pltpu.VMEM((C,), jnp.int32),
                   pltpu.VMEM((C,), jnp.float32),
                   pltpu.VMEM((C,), jnp.float32),
                   pltpu.VMEM((NPAD + 16,), jnp.float32),
                   pltpu.SemaphoreType.DMA),
)
def _s2(attn, dst, maxpart, ex_out, denpart_out,
        segmax, tmp, idx_d, attn_buf, ex_buf, denloc, sem):
    w = _wid()
    mask0 = lax.iota(jnp.int32, 16) == 0
    pltpu.sync_copy(maxpart.at[0], segmax)

    def merge(j, _):
        pltpu.sync_copy(maxpart.at[j], tmp)

        def mrow(i, _):
            sl = pl.ds(i * 16, 16)
            segmax[sl] = jnp.maximum(segmax[sl], tmp[sl])
            return 0
        lax.fori_loop(0, NPAD // 16, mrow, 0)
        return 0
    lax.fori_loop(1, NW, merge, 0)

    def fix(i, _):
        sl = pl.ds(i * 16, 16)
        v = segmax[sl]
        segmax[sl] = jnp.where(v < NEG, jnp.zeros((16,), jnp.float32), v)
        return 0
    lax.fori_loop(0, NPAD // 16, fix, 0)

    _fill(denloc, NPAD + 16, 0.0)

    def chunk(k, _):
        base = w * EPW + k * C

        @pl.when(base < E)
        def _():
            pltpu.sync_copy(dst.at[pl.ds(base, C)], idx_d)
            pltpu.sync_copy(attn.at[pl.ds(base, C)], attn_buf)

            def grp(i, _):
                sl = pl.ds(i * 16, 16)
                dv = idx_d[sl]
                m = plsc.load_gather(segmax, [dv])
                ex_buf[sl] = jnp.exp(attn_buf[sl] - m)
                return 0
            lax.fori_loop(0, C // 16, grp, 0)

            def acc(gi, _):
                dv = idx_d[pl.ds(gi * 16, 16)]
                exv = ex_buf[pl.ds(gi * 16, 16)]
                for e in range(16):
                    d = dv[e]
                    old = denloc[pl.ds(d, 16)]
                    denloc[pl.ds(d, 16)] = jnp.where(
                        mask0, old + exv[e], old)
                return 0
            lax.fori_loop(0, C // 16, acc, 0)
            pltpu.sync_copy(ex_buf, ex_out.at[pl.ds(base, C)])
        return 0
    lax.fori_loop(0, NCHUNK, chunk, 0)
    pltpu.sync_copy(denloc.at[pl.ds(0, NPAD)], denpart_out.at[w])


@functools.partial(
    pl.kernel, mesh=_mesh,
    compiler_params=pltpu.CompilerParams(needs_layout_passes=False),
    out_type=jax.ShapeDtypeStruct((2, NPAD, D), jnp.float32),
    scratch_types=(pltpu.VMEM((NPAD,), jnp.float32),
                   pltpu.VMEM((NPAD,), jnp.float32),
                   pltpu.VMEM((C,), jnp.int32),
                   pltpu.VMEM((C,), jnp.int32),
                   pltpu.VMEM((C,), jnp.float32),
                   pltpu.VMEM((C,), jnp.float32),
                   pltpu.VMEM((C, D), jnp.float32),
                   pltpu.VMEM_SHARED((NPAD, D), jnp.float32),
                   pltpu.SemaphoreType.DMA),
)
def _s3(nh, src, dst, ex, denpart, nzpart_out,
        denom, tmp, idx_s, idx_d, ex_buf, a_buf, rows, nzacc, sem):
    w = _wid()
    c = lax.axis_index("c")
    s = lax.axis_index("s")
    pltpu.sync_copy(denpart.at[0], denom)

    def merge(j, _):
        pltpu.sync_copy(denpart.at[j], tmp)

        def mrow(i, _):
            sl = pl.ds(i * 16, 16)
            denom[sl] = denom[sl] + tmp[sl]
            return 0
        lax.fori_loop(0, NPAD // 16, mrow, 0)
        return 0
    lax.fori_loop(1, NW, merge, 0)

    def zrow(i, _):
        for j in range(8):
            rows[i, pl.ds(j * 16, 16)] = jnp.zeros((16,), jnp.float32)
        return 0
    lax.fori_loop(0, C, zrow, 0)
    for m in range(5):
        pltpu.sync_copy(rows, nzacc.at[pl.ds(s * 640 + m * C, C)])
    plsc.subcore_barrier()

    def chunk(k, _):
        base = w * EPW + k * C

        @pl.when(base < E)
        def _():
            pltpu.sync_copy(src.at[pl.ds(base, C)], idx_s)
            pltpu.sync_copy(dst.at[pl.ds(base, C)], idx_d)
            pltpu.sync_copy(ex.at[pl.ds(base, C)], ex_buf)
            pltpu.async_copy(nh.at[idx_s], rows, sem).wait()

            def grp(i, _):
                sl = pl.ds(i * 16, 16)
                dv = idx_d[sl]
                den = plsc.load_gather(denom, [dv])
                a_buf[sl] = ex_buf[sl] / den
                return 0
            lax.fori_loop(0, C // 16, grp, 0)

            def scale(gi, _):
                av16 = a_buf[pl.ds(gi * 16, 16)]
                for e in range(16):
                    i = gi * 16 + e
                    av = lax.broadcast_in_dim(av16[e], (16,), ())
                    for j in range(8):
                        sl = pl.ds(j * 16, 16)
                        rows[i, sl] = rows[i, sl] * av
                return 0
            lax.fori_loop(0, C // 16, scale, 0)
            pltpu.sync_copy(rows, nzacc.at[idx_d], add=True)
        return 0
    lax.fori_loop(0, NCHUNK, chunk, 0)
    plsc.subcore_barrier()
    for m in range(5):
        sl = pl.ds(s * 640 + m * C, C)
        pltpu.sync_copy(nzacc.at[sl], nzpart_out.at[c, sl])


@functools.partial(
    pl.kernel, mesh=_mesh,
    compiler_params=pltpu.CompilerParams(needs_layout_passes=False),
    out_type=jax.ShapeDtypeStruct((E, D), jnp.float32),
    scratch_types=(pltpu.VMEM((C,), jnp.int32),
                   pltpu.VMEM((C,), jnp.int32),
                   pltpu.VMEM((C, D), jnp.float32),
                   pltpu.VMEM((C, D), jnp.float32),
                   pltpu.SemaphoreType.DMA),
)
def _s4(nz, src, dst, g_out, idx_s, idx_d, arows, brows, sem):
    w = _wid()

    def chunk(k, _):
        base = w * EPW + k * C

        @pl.when(base < E)
        def _():
            pltpu.sync_copy(src.at[pl.ds(base, C)], idx_s)
            pltpu.sync_copy(dst.at[pl.ds(base, C)], idx_d)
            c1 = pltpu.async_copy(nz.at[idx_s], arows, sem)
            c2 = pltpu.async_copy(nz.at[idx_d], brows, sem)
            c1.wait()
            c2.wait()

            def sub(i, _):
                for j in range(8):
                    sl = pl.ds(j * 16, 16)
                    arows[i, sl] = arows[i, sl] - brows[i, sl]
                return 0
            lax.fori_loop(0, C, sub, 0)
            pltpu.sync_copy(arows, g_out.at[pl.ds(base, C)])
        return 0
    lax.fori_loop(0, NCHUNK, chunk, 0)


def _t0_body(p0, p1, nz_out):
    nz_out[...] = p0[0:N, :] + p1[0:N, :]


def _stats_body(a, b, eps, w1, b1, w2, b2, y_out, s_out, q_out):
    x = (1.0 + eps[...]) * a[...] + b[...]
    h = jnp.maximum(jnp.dot(x, w1[...], preferred_element_type=jnp.float32)
                    + b1[...], 0.0)
    y = jnp.dot(h, w2[...], preferred_element_type=jnp.float32) + b2[...]
    y_out[...] = y

    @pl.when(pl.program_id(0) == 0)
    def _():
        s_out[...] = jnp.zeros_like(s_out)
        q_out[...] = jnp.zeros_like(q_out)
    s_out[...] += jnp.sum(y, axis=0, keepdims=True)
    q_out[...] += jnp.sum(y * y, axis=0, keepdims=True)


def _norm_body(n_rows, y, s, q, gamma, beta, out):
    mu = s[...] / n_rows
    var = q[...] / n_rows - mu * mu
    inv = lax.rsqrt(var + 1e-5)
    out[...] = (y[...] - mu) * inv * gamma[...] + beta[...]


def _mlp_bn(x_a, x_b, eps, w1, b1, w2, b2, gamma, beta, n_rows, blk):
    nblk = n_rows // blk
    row_spec = pl.BlockSpec((blk, D), lambda i: (i, 0))
    vec_spec = pl.BlockSpec((1, D), lambda i: (0, 0))
    mat_spec = pl.BlockSpec((D, D), lambda i: (0, 0))
    y, ssum, qsum = pl.pallas_call(
        _stats_body,
        grid=(nblk,),
        in_specs=[row_spec, row_spec, vec_spec, mat_spec, vec_spec,
                  mat_spec, vec_spec],
        out_specs=[row_spec, vec_spec, vec_spec],
        out_shape=[jax.ShapeDtypeStruct((n_rows, D), jnp.float32),
                   jax.ShapeDtypeStruct((1, D), jnp.float32),
                   jax.ShapeDtypeStruct((1, D), jnp.float32)],
    )(x_a, x_b, eps, w1, b1, w2, b2)
    out = pl.pallas_call(
        functools.partial(_norm_body, float(n_rows)),
        grid=(nblk,),
        in_specs=[row_spec, vec_spec, vec_spec, vec_spec, vec_spec],
        out_specs=row_spec,
        out_shape=jax.ShapeDtypeStruct((n_rows, D), jnp.float32),
    )(y, ssum, qsum, gamma, beta)
    return out


def kernel(nh, eh, edge_index, nf_W1, nf_b1, nf_W2, nf_b2, nf_eps,
           nf_gamma, nf_beta, ef_W1, ef_b1, ef_W2, ef_b2, ef_eps,
           ef_gamma, ef_beta):
    src = edge_index[0]
    dst = edge_index[1]

    attn, maxpart = _s1(nh, eh, src, dst)
    ex, denpart = _s2(attn, dst, maxpart)
    nzpart = _s3(nh, src, dst, ex, denpart)

    nz = pl.pallas_call(
        _t0_body,
        out_shape=jax.ShapeDtypeStruct((N, D), jnp.float32),
    )(nzpart[0], nzpart[1])

    r = lambda v: v.reshape(1, D)
    n_h = _mlp_bn(nh, nz, r(nf_eps), nf_W1, r(nf_b1), nf_W2, r(nf_b2),
                  r(nf_gamma), r(nf_beta), N, 2000)

    g = _s4(nz, src, dst)
    e_h = _mlp_bn(eh, g, r(ef_eps), ef_W1, r(ef_b1), ef_W2, r(ef_b2),
                  r(ef_gamma), r(ef_beta), E, 2000)
    return (n_h, e_h)


# speedup vs baseline: 4.7331x; 1.2890x over previous
import functools
import jax
import jax.numpy as jnp
from jax import lax
from jax.experimental import pallas as pl
from jax.experimental.pallas import tpu as pltpu
from jax.experimental.pallas import tpu_sc as plsc

N = 10000
E = 320000
D = 128
NPAD = 10240
NW = 32
EPW = 10240
C = 128
NCHUNK = EPW // C
NEG = -3.0e38

_mesh = plsc.VectorSubcoreMesh(core_axis_name="c", subcore_axis_name="s")
_params = pltpu.CompilerParams(needs_layout_passes=False)


def _wid():
    return lax.axis_index("s") * 2 + lax.axis_index("c")


def _fill(ref, n, val):
    def body(i, _):
        for u in range(4):
            ref[pl.ds(i * 64 + u * 16, 16)] = jnp.full((16,), val,
                                                       jnp.float32)
        return 0
    lax.fori_loop(0, n // 64, body, 0)


MS = 2560


def _merge(part, acc, tmp0, tmp1, sem, op):
    nstrip = NPAD // MS
    total = NW * nstrip

    def src_of(t):
        return part.at[t // nstrip, pl.ds((t % nstrip) * MS, MS)]

    def fetch(t, tmp):
        @pl.when(jnp.asarray(t) < total)
        def _():
            pltpu.async_copy(src_of(t), tmp, sem)

    def consume(t, tmp):
        pltpu.make_async_copy(src_of(t), tmp, sem).wait()
        boff = (t % nstrip) * MS

        def mrow(i, _):
            for u in range(4):
                o = i * 64 + u * 16
                sl_a = pl.ds(boff + o, 16)
                acc[sl_a] = op(acc[sl_a], tmp[pl.ds(o, 16)])
            return 0
        lax.fori_loop(0, MS // 64, mrow, 0)

    fetch(0, tmp0)

    def pair(p, _):
        t0 = 2 * p
        fetch(t0 + 1, tmp1)
        consume(t0, tmp0)
        fetch(t0 + 2, tmp0)
        consume(t0 + 1, tmp1)
        return 0
    lax.fori_loop(0, total // 2, pair, 0)


def _real(k, base):
    return jnp.logical_and(jnp.asarray(k) < NCHUNK, base < E)


@functools.partial(
    pl.kernel, mesh=_mesh, compiler_params=_params,
    out_type=(jax.ShapeDtypeStruct((E,), jnp.float32),
              jax.ShapeDtypeStruct((NW, NPAD), jnp.float32)),
    scratch_types=(pltpu.VMEM((C,), jnp.int32),
                   pltpu.VMEM((C,), jnp.int32),
                   pltpu.VMEM((C,), jnp.int32),
                   pltpu.VMEM((C,), jnp.int32),
                   pltpu.VMEM((C, D), jnp.float32),
                   pltpu.VMEM((C, D), jnp.float32),
                   pltpu.VMEM((C, D), jnp.float32),
                   pltpu.VMEM((C, D), jnp.float32),
                   pltpu.VMEM((C, D), jnp.float32),
                   pltpu.VMEM((C, D), jnp.float32),
                   pltpu.VMEM((C,), jnp.float32),
                   pltpu.VMEM((C,), jnp.float32),
                   pltpu.VMEM((NPAD + 16,), jnp.float32),
                   pltpu.SemaphoreType.DMA,
                   pltpu.SemaphoreType.DMA,
                   pltpu.SemaphoreType.DMA,
                   pltpu.SemaphoreType.DMA),
)
def _s1(nh, eh, src, dst, attn_out, maxpart_out,
        ixs0, ixs1, ixd0, ixd1, sr0, sr1, dr0, dr1, er0, er1, ab0, ab1,
        maxloc, semA, semB, semW0, semW1):
    w = _wid()
    _fill(maxloc, NPAD + 16, NEG)
    lane = lax.iota(jnp.int32, 16)
    mask0 = lane == 0
    A = (ixs0, ixd0, sr0, dr0, er0, ab0, semA, semW0)
    B = (ixs1, ixd1, sr1, dr1, er1, ab1, semB, semW1)

    def start(k, buf):
        ixs, ixd, sr, dr, er, ab, sem, semw = buf
        base = w * EPW + k * C

        @pl.when(_real(k, base))
        def _():
            pltpu.sync_copy(src.at[pl.ds(base, C)], ixs)
            pltpu.sync_copy(dst.at[pl.ds(base, C)], ixd)
            pltpu.async_copy(nh.at[ixs], sr, sem)
            pltpu.async_copy(nh.at[ixd], dr, sem)
            pltpu.async_copy(eh.at[pl.ds(base, C)], er, sem)

    def step(k, buf):
        ixs, ixd, sr, dr, er, ab, sem, semw = buf
        base = w * EPW + k * C

        @pl.when(base < E)
        def _():
            pltpu.make_async_copy(nh.at[ixs], sr, sem).wait()
            pltpu.make_async_copy(nh.at[ixd], dr, sem).wait()
            pltpu.make_async_copy(eh.at[pl.ds(base, C)], er, sem).wait()

            @pl.when(jnp.asarray(k) >= 2)
            def _():
                pltpu.make_async_copy(ab, attn_out.at[pl.ds(base, C)],
                                      semw).wait()

            def grp(gi, _):
                vec = jnp.zeros((16,), jnp.float32)
                dv = ixd[pl.ds(gi * 16, 16)]
                for e in range(16):
                    i = gi * 16 + e
                    acc = jnp.zeros((16,), jnp.float32)
                    for j in range(8):
                        sl = pl.ds(j * 16, 16)
                        acc += (sr[i, sl] + er[i, sl]) * dr[i, sl]
                    v = jnp.sum(acc)
                    vec = jnp.where(lane == e, v, vec)
                    d = dv[e]
                    old = maxloc[pl.ds(d, 16)]
                    maxloc[pl.ds(d, 16)] = jnp.where(
                        mask0, jnp.maximum(old, v), old)
                ab[pl.ds(gi * 16, 16)] = vec
                return 0
            lax.fori_loop(0, C // 16, grp, 0)
            pltpu.async_copy(ab, attn_out.at[pl.ds(base, C)], semw)

    start(0, A)

    def pipe(g, _):
        k0 = 2 * g
        start(k0 + 1, B)
        step(k0, A)
        start(k0 + 2, A)
        step(k0 + 1, B)
        return 0
    lax.fori_loop(0, NCHUNK // 2, pipe, 0)
    pltpu.make_async_copy(ab0, attn_out.at[pl.ds(0, C)], semW0).wait()
    pltpu.make_async_copy(ab1, attn_out.at[pl.ds(0, C)], semW1).wait()
    pltpu.sync_copy(maxloc.at[pl.ds(0, NPAD)], maxpart_out.at[w])


@functools.partial(
    pl.kernel, mesh=_mesh, compiler_params=_params,
    out_type=(jax.ShapeDtypeStruct((E,), jnp.float32),
              jax.ShapeDtypeStruct((NW, NPAD), jnp.float32)),
    scratch_types=(pltpu.VMEM((NPAD,), jnp.float32),
                   pltpu.VMEM((MS,), jnp.float32),
                   pltpu.VMEM((MS,), jnp.float32),
                   pltpu.VMEM((NPAD + 16,), jnp.float32),
                   pltpu.VMEM((C,), jnp.int32),
                   pltpu.VMEM((C,), jnp.int32),
                   pltpu.VMEM((C,), jnp.float32),
                   pltpu.VMEM((C,), jnp.float32),
                   pltpu.VMEM((C,), jnp.float32),
                   pltpu.VMEM((C,), jnp.float32),
                   pltpu.SemaphoreType.DMA,
                   pltpu.SemaphoreType.DMA,
                   pltpu.SemaphoreType.DMA,
                   pltpu.SemaphoreType.DMA),
)
def _s2(attn, dst, maxpart, ex_out, denpart_out,
        segmax, tmp0, tmp1, denloc, ixd0, ixd1, ab0, ab1, eb0, eb1,
        semA, semB, semW0, semW1):
    w = _wid()
    mask0 = lax.iota(jnp.int32, 16) == 0
    _fill(segmax, NPAD, NEG)
    _merge(maxpart, segmax, tmp0, tmp1, semA, jnp.maximum)

    def fix(i, _):
        for u in range(4):
            sl = pl.ds(i * 64 + u * 16, 16)
            v = segmax[sl]
            segmax[sl] = jnp.where(v < NEG, jnp.zeros((16,), jnp.float32),
                                   v)
        return 0
    lax.fori_loop(0, NPAD // 64, fix, 0)

    _fill(denloc, NPAD + 16, 0.0)
    A = (ixd0, ab0, eb0, semA, semW0)
    B = (ixd1, ab1, eb1, semB, semW1)

    def start(k, buf):
        ixd, ab, eb, sem, semw = buf
        base = w * EPW + k * C

        @pl.when(_real(k, base))
        def _():
            pltpu.async_copy(dst.at[pl.ds(base, C)], ixd, sem)
            pltpu.async_copy(attn.at[pl.ds(base, C)], ab, sem)

    def step(k, buf):
        ixd, ab, eb, sem, semw = buf
        base = w * EPW + k * C

        @pl.when(base < E)
        def _():
            pltpu.make_async_copy(dst.at[pl.ds(base, C)], ixd, sem).wait()
            pltpu.make_async_copy(attn.at[pl.ds(base, C)], ab, sem).wait()

            @pl.when(jnp.asarray(k) >= 2)
            def _():
                pltpu.make_async_copy(eb, ex_out.at[pl.ds(base, C)],
                                      semw).wait()

            def grp(i, _):
                sl = pl.ds(i * 16, 16)
                dv = ixd[sl]
                m = plsc.load_gather(segmax, [dv])
                eb[sl] = jnp.exp(ab[sl] - m)
                return 0
            lax.fori_loop(0, C // 16, grp, 0)

            def acc(gi, _):
                dv = ixd[pl.ds(gi * 16, 16)]
                exv = eb[pl.ds(gi * 16, 16)]
                for e in range(16):
                    d = dv[e]
                    old = denloc[pl.ds(d, 16)]
                    denloc[pl.ds(d, 16)] = jnp.where(
                        mask0, old + exv[e], old)
                return 0
            lax.fori_loop(0, C // 16, acc, 0)
            pltpu.async_copy(eb, ex_out.at[pl.ds(base, C)], semw)

    start(0, A)

    def pipe(g, _):
        k0 = 2 * g
        start(k0 + 1, B)
        step(k0, A)
        start(k0 + 2, A)
        step(k0 + 1, B)
        return 0
    lax.fori_loop(0, NCHUNK // 2, pipe, 0)
    pltpu.make_async_copy(eb0, ex_out.at[pl.ds(0, C)], semW0).wait()
    pltpu.make_async_copy(eb1, ex_out.at[pl.ds(0, C)], semW1).wait()
    pltpu.sync_copy(denloc.at[pl.ds(0, NPAD)], denpart_out.at[w])


@functools.partial(
    pl.kernel, mesh=_mesh, compiler_params=_params,
    out_type=jax.ShapeDtypeStruct((2, NPAD, D), jnp.float32),
    scratch_types=(pltpu.VMEM((NPAD,), jnp.float32),
                   pltpu.VMEM((MS,), jnp.float32),
                   pltpu.VMEM((MS,), jnp.float32),
                   pltpu.VMEM((C,), jnp.int32),
                   pltpu.VMEM((C,), jnp.int32),
                   pltpu.VMEM((C,), jnp.float32),
                   pltpu.VMEM((C,), jnp.float32),
                   pltpu.VMEM((C, D), jnp.float32),
                   pltpu.VMEM_SHARED((NPAD, D), jnp.float32),
                   pltpu.SemaphoreType.DMA),
)
def _s3(nh, src, dst, ex, denpart, nzpart_out,
        denom, tmp0, tmp1, ixs, ixd, eb, ab, rows, nzacc, sem):
    w = _wid()
    c = lax.axis_index("c")
    s = lax.axis_index("s")
    _fill(denom, NPAD, 0.0)
    _merge(denpart, denom, tmp0, tmp1, sem, jnp.add)

    def zrow(i, _):
        for j in range(8):
            rows[i, pl.ds(j * 16, 16)] = jnp.zeros((16,), jnp.float32)
        return 0
    lax.fori_loop(0, C, zrow, 0)
    for m in range(5):
        pltpu.sync_copy(rows, nzacc.at[pl.ds(s * 640 + m * C, C)])
    plsc.subcore_barrier()

    def chunk(k, _):
        base = w * EPW + k * C

        @pl.when(base < E)
        def _():
            pltpu.sync_copy(src.at[pl.ds(base, C)], ixs)
            pltpu.sync_copy(dst.at[pl.ds(base, C)], ixd)
            pltpu.sync_copy(ex.at[pl.ds(base, C)], eb)
            pltpu.async_copy(nh.at[ixs], rows, sem).wait()

            def grp(i, _):
                sl = pl.ds(i * 16, 16)
                dv = ixd[sl]
                den = plsc.load_gather(denom, [dv])
                ab[sl] = eb[sl] / den
                return 0
            lax.fori_loop(0, C // 16, grp, 0)

            def scale(gi, _):
                av16 = ab[pl.ds(gi * 16, 16)]
                for e in range(16):
                    i = gi * 16 + e
                    av = lax.broadcast_in_dim(av16[e], (16,), ())
                    for j in range(8):
                        sl = pl.ds(j * 16, 16)
                        rows[i, sl] = rows[i, sl] * av
                return 0
            lax.fori_loop(0, C // 16, scale, 0)
            pltpu.sync_copy(rows, nzacc.at[ixd], add=True)
        return 0
    lax.fori_loop(0, NCHUNK, chunk, 0)
    plsc.subcore_barrier()
    for m in range(5):
        sl = pl.ds(s * 640 + m * C, C)
        pltpu.sync_copy(nzacc.at[sl], nzpart_out.at[c, sl])


@functools.partial(
    pl.kernel, mesh=_mesh, compiler_params=_params,
    out_type=jax.ShapeDtypeStruct((E, D), jnp.float32),
    scratch_types=(pltpu.VMEM((C,), jnp.int32),
                   pltpu.VMEM((C,), jnp.int32),
                   pltpu.VMEM((C,), jnp.int32),
                   pltpu.VMEM((C,), jnp.int32),
                   pltpu.VMEM((C, D), jnp.float32),
                   pltpu.VMEM((C, D), jnp.float32),
                   pltpu.VMEM((C, D), jnp.float32),
                   pltpu.VMEM((C, D), jnp.float32),
                   pltpu.SemaphoreType.DMA,
                   pltpu.SemaphoreType.DMA,
                   pltpu.SemaphoreType.DMA,
                   pltpu.SemaphoreType.DMA),
)
def _s4(nz, src, dst, g_out,
        ixs0, ixs1, ixd0, ixd1, ar0, ar1, br0, br1,
        semA, semB, semW0, semW1):
    w = _wid()
    A = (ixs0, ixd0, ar0, br0, semA, semW0)
    B = (ixs1, ixd1, ar1, br1, semB, semW1)

    def start(k, buf):
        ixs, ixd, ar, br, sem, semw = buf
        base = w * EPW + k * C

        @pl.when(_real(k, base))
        def _():
            @pl.when(jnp.asarray(k) >= 2)
            def _():
                pltpu.make_async_copy(ar, g_out.at[pl.ds(base, C)],
                                      semw).wait()
            pltpu.sync_copy(src.at[pl.ds(base, C)], ixs)
            pltpu.sync_copy(dst.at[pl.ds(base, C)], ixd)
            pltpu.async_copy(nz.at[ixs], ar, sem)
            pltpu.async_copy(nz.at[ixd], br, sem)

    def step(k, buf):
        ixs, ixd, ar, br, sem, semw = buf
        base = w * EPW + k * C

        @pl.when(base < E)
        def _():
            pltpu.make_async_copy(nz.at[ixs], ar, sem).wait()
            pltpu.make_async_copy(nz.at[ixd], br, sem).wait()

            def sub(i, _):
                for j in range(8):
                    sl = pl.ds(j * 16, 16)
                    ar[i, sl] = ar[i, sl] - br[i, sl]
                return 0
            lax.fori_loop(0, C, sub, 0)
            pltpu.async_copy(ar, g_out.at[pl.ds(base, C)], semw)

    start(0, A)

    def pipe(g, _):
        k0 = 2 * g
        start(k0 + 1, B)
        step(k0, A)
        start(k0 + 2, A)
        step(k0 + 1, B)
        return 0
    lax.fori_loop(0, NCHUNK // 2, pipe, 0)
    pltpu.make_async_copy(ar0, g_out.at[pl.ds(0, C)], semW0).wait()
    pltpu.make_async_copy(ar1, g_out.at[pl.ds(0, C)], semW1).wait()


def _t0_body(p0, p1, nz_out):
    nz_out[...] = p0[0:N, :] + p1[0:N, :]


def _stats_body(a, b, eps, w1, b1, w2, b2, y_out, s_out, q_out):
    x = (1.0 + eps[...]) * a[...] + b[...]
    h = jnp.maximum(jnp.dot(x, w1[...], preferred_element_type=jnp.float32)
                    + b1[...], 0.0)
    y = jnp.dot(h, w2[...], preferred_element_type=jnp.float32) + b2[...]
    y_out[...] = y

    @pl.when(pl.program_id(0) == 0)
    def _():
        s_out[...] = jnp.zeros_like(s_out)
        q_out[...] = jnp.zeros_like(q_out)
    s_out[...] += jnp.sum(y, axis=0, keepdims=True)
    q_out[...] += jnp.sum(y * y, axis=0, keepdims=True)


def _norm_body(n_rows, y, s, q, gamma, beta, out):
    mu = s[...] / n_rows
    var = q[...] / n_rows - mu * mu
    inv = lax.rsqrt(var + 1e-5)
    out[...] = (y[...] - mu) * inv * gamma[...] + beta[...]


def _mlp_bn(x_a, x_b, eps, w1, b1, w2, b2, gamma, beta, n_rows, blk):
    nblk = n_rows // blk
    row_spec = pl.BlockSpec((blk, D), lambda i: (i, 0))
    vec_spec = pl.BlockSpec((1, D), lambda i: (0, 0))
    mat_spec = pl.BlockSpec((D, D), lambda i: (0, 0))
    y, ssum, qsum = pl.pallas_call(
        _stats_body,
        grid=(nblk,),
        in_specs=[row_spec, row_spec, vec_spec, mat_spec, vec_spec,
                  mat_spec, vec_spec],
        out_specs=[row_spec, vec_spec, vec_spec],
        out_shape=[jax.ShapeDtypeStruct((n_rows, D), jnp.float32),
                   jax.ShapeDtypeStruct((1, D), jnp.float32),
                   jax.ShapeDtypeStruct((1, D), jnp.float32)],
    )(x_a, x_b, eps, w1, b1, w2, b2)
    out = pl.pallas_call(
        functools.partial(_norm_body, float(n_rows)),
        grid=(nblk,),
        in_specs=[row_spec, vec_spec, vec_spec, vec_spec, vec_spec],
        out_specs=row_spec,
        out_shape=jax.ShapeDtypeStruct((n_rows, D), jnp.float32),
    )(y, ssum, qsum, gamma, beta)
    return out


def kernel(nh, eh, edge_index, nf_W1, nf_b1, nf_W2, nf_b2, nf_eps,
           nf_gamma, nf_beta, ef_W1, ef_b1, ef_W2, ef_b2, ef_eps,
           ef_gamma, ef_beta):
    src = edge_index[0]
    dst = edge_index[1]

    attn, maxpart = _s1(nh, eh, src, dst)
    ex, denpart = _s2(attn, dst, maxpart)
    nzpart = _s3(nh, src, dst, ex, denpart)

    nz = pl.pallas_call(
        _t0_body,
        out_shape=jax.ShapeDtypeStruct((N, D), jnp.float32),
    )(nzpart[0], nzpart[1])

    r = lambda v: v.reshape(1, D)
    n_h = _mlp_bn(nh, nz, r(nf_eps), nf_W1, r(nf_b1), nf_W2, r(nf_b2),
                  r(nf_gamma), r(nf_beta), N, 2000)

    g = _s4(nz, src, dst)
    e_h = _mlp_bn(eh, g, r(ef_eps), ef_W1, r(ef_b1), ef_W2, r(ef_b2),
                  r(ef_gamma), r(ef_beta), E, 2000)
    return (n_h, e_h)


# speedup vs baseline: 4.7528x; 1.0042x over previous
import functools
import jax
import jax.numpy as jnp
from jax import lax
from jax.experimental import pallas as pl
from jax.experimental.pallas import tpu as pltpu
from jax.experimental.pallas import tpu_sc as plsc

N = 10000
E = 320000
D = 128
NPAD = 10240
NW = 32
EPW = 10240
C = 128
NCHUNK = EPW // C
NEG = -3.0e38

_mesh = plsc.VectorSubcoreMesh(core_axis_name="c", subcore_axis_name="s")
_params = pltpu.CompilerParams(needs_layout_passes=False)


def _wid():
    return lax.axis_index("s") * 2 + lax.axis_index("c")


def _fill(ref, n, val):
    def body(i, _):
        for u in range(4):
            ref[pl.ds(i * 64 + u * 16, 16)] = jnp.full((16,), val,
                                                       jnp.float32)
        return 0
    lax.fori_loop(0, n // 64, body, 0)


MS = 2560


def _merge(part, acc, tmp0, tmp1, sem, op):
    nstrip = NPAD // MS
    total = NW * nstrip

    def src_of(t):
        return part.at[t // nstrip, pl.ds((t % nstrip) * MS, MS)]

    def fetch(t, tmp):
        @pl.when(jnp.asarray(t) < total)
        def _():
            pltpu.async_copy(src_of(t), tmp, sem)

    def consume(t, tmp):
        pltpu.make_async_copy(src_of(t), tmp, sem).wait()
        boff = (t % nstrip) * MS

        def mrow(i, _):
            for u in range(4):
                o = i * 64 + u * 16
                sl_a = pl.ds(boff + o, 16)
                acc[sl_a] = op(acc[sl_a], tmp[pl.ds(o, 16)])
            return 0
        lax.fori_loop(0, MS // 64, mrow, 0)

    fetch(0, tmp0)

    def pair(p, _):
        t0 = 2 * p
        fetch(t0 + 1, tmp1)
        consume(t0, tmp0)
        fetch(t0 + 2, tmp0)
        consume(t0 + 1, tmp1)
        return 0
    lax.fori_loop(0, total // 2, pair, 0)


def _real(k, base):
    return jnp.logical_and(jnp.asarray(k) < NCHUNK, base < E)


@functools.partial(
    pl.kernel, mesh=_mesh, compiler_params=_params,
    out_type=(jax.ShapeDtypeStruct((E,), jnp.float32),
              jax.ShapeDtypeStruct((NW, NPAD), jnp.float32)),
    scratch_types=(pltpu.VMEM((C,), jnp.int32),
                   pltpu.VMEM((C,), jnp.int32),
                   pltpu.VMEM((C,), jnp.int32),
                   pltpu.VMEM((C,), jnp.int32),
                   pltpu.VMEM((C, D), jnp.float32),
                   pltpu.VMEM((C, D), jnp.float32),
                   pltpu.VMEM((C, D), jnp.float32),
                   pltpu.VMEM((C, D), jnp.float32),
                   pltpu.VMEM((C, D), jnp.float32),
                   pltpu.VMEM((C, D), jnp.float32),
                   pltpu.VMEM((C,), jnp.float32),
                   pltpu.VMEM((C,), jnp.float32),
                   pltpu.VMEM((NPAD + 16,), jnp.float32),
                   pltpu.SemaphoreType.DMA,
                   pltpu.SemaphoreType.DMA,
                   pltpu.SemaphoreType.DMA,
                   pltpu.SemaphoreType.DMA),
)
def _s1(nh, eh, src, dst, attn_out, maxpart_out,
        ixs0, ixs1, ixd0, ixd1, sr0, sr1, dr0, dr1, er0, er1, ab0, ab1,
        maxloc, semA, semB, semW0, semW1):
    w = _wid()
    _fill(maxloc, NPAD + 16, NEG)
    lane = lax.iota(jnp.int32, 16)
    mask0 = lane == 0
    A = (ixs0, ixd0, sr0, dr0, er0, ab0, semA, semW0)
    B = (ixs1, ixd1, sr1, dr1, er1, ab1, semB, semW1)

    def start(k, buf):
        ixs, ixd, sr, dr, er, ab, sem, semw = buf
        base = w * EPW + k * C

        @pl.when(_real(k, base))
        def _():
            pltpu.sync_copy(src.at[pl.ds(base, C)], ixs)
            pltpu.sync_copy(dst.at[pl.ds(base, C)], ixd)
            pltpu.async_copy(nh.at[ixs], sr, sem)
            pltpu.async_copy(nh.at[ixd], dr, sem)
            pltpu.async_copy(eh.at[pl.ds(base, C)], er, sem)

    def step(k, buf):
        ixs, ixd, sr, dr, er, ab, sem, semw = buf
        base = w * EPW + k * C

        @pl.when(base < E)
        def _():
            pltpu.make_async_copy(nh.at[ixs], sr, sem).wait()
            pltpu.make_async_copy(nh.at[ixd], dr, sem).wait()
            pltpu.make_async_copy(eh.at[pl.ds(base, C)], er, sem).wait()

            @pl.when(jnp.asarray(k) >= 2)
            def _():
                pltpu.make_async_copy(ab, attn_out.at[pl.ds(base, C)],
                                      semw).wait()

            def grp(gi, _):
                vec = jnp.zeros((16,), jnp.float32)
                dv = ixd[pl.ds(gi * 16, 16)]
                for e in range(16):
                    i = gi * 16 + e
                    acc = jnp.zeros((16,), jnp.float32)
                    for j in range(8):
                        sl = pl.ds(j * 16, 16)
                        acc += (sr[i, sl] + er[i, sl]) * dr[i, sl]
                    v = jnp.sum(acc)
                    vec = jnp.where(lane == e, v, vec)
                    d = dv[e]
                    old = maxloc[pl.ds(d, 16)]
                    maxloc[pl.ds(d, 16)] = jnp.where(
                        mask0, jnp.maximum(old, v), old)
                ab[pl.ds(gi * 16, 16)] = vec
                return 0
            lax.fori_loop(0, C // 16, grp, 0)
            pltpu.async_copy(ab, attn_out.at[pl.ds(base, C)], semw)

    start(0, A)

    def pipe(g, _):
        k0 = 2 * g
        start(k0 + 1, B)
        step(k0, A)
        start(k0 + 2, A)
        step(k0 + 1, B)
        return 0
    lax.fori_loop(0, NCHUNK // 2, pipe, 0)
    pltpu.make_async_copy(ab0, attn_out.at[pl.ds(0, C)], semW0).wait()
    pltpu.make_async_copy(ab1, attn_out.at[pl.ds(0, C)], semW1).wait()
    pltpu.sync_copy(maxloc.at[pl.ds(0, NPAD)], maxpart_out.at[w])


@functools.partial(
    pl.kernel, mesh=_mesh, compiler_params=_params,
    out_type=(jax.ShapeDtypeStruct((E,), jnp.float32),
              jax.ShapeDtypeStruct((NW, NPAD), jnp.float32)),
    scratch_types=(pltpu.VMEM((NPAD,), jnp.float32),
                   pltpu.VMEM((MS,), jnp.float32),
                   pltpu.VMEM((MS,), jnp.float32),
                   pltpu.VMEM((NPAD + 16,), jnp.float32),
                   pltpu.VMEM((C,), jnp.int32),
                   pltpu.VMEM((C,), jnp.int32),
                   pltpu.VMEM((C,), jnp.float32),
                   pltpu.VMEM((C,), jnp.float32),
                   pltpu.VMEM((C,), jnp.float32),
                   pltpu.VMEM((C,), jnp.float32),
                   pltpu.SemaphoreType.DMA,
                   pltpu.SemaphoreType.DMA,
                   pltpu.SemaphoreType.DMA,
                   pltpu.SemaphoreType.DMA),
)
def _s2(attn, dst, maxpart, ex_out, denpart_out,
        segmax, tmp0, tmp1, denloc, ixd0, ixd1, ab0, ab1, eb0, eb1,
        semA, semB, semW0, semW1):
    w = _wid()
    mask0 = lax.iota(jnp.int32, 16) == 0
    _fill(segmax, NPAD, NEG)
    _merge(maxpart, segmax, tmp0, tmp1, semA, jnp.maximum)

    def fix(i, _):
        for u in range(4):
            sl = pl.ds(i * 64 + u * 16, 16)
            v = segmax[sl]
            segmax[sl] = jnp.where(v < NEG, jnp.zeros((16,), jnp.float32),
                                   v)
        return 0
    lax.fori_loop(0, NPAD // 64, fix, 0)

    _fill(denloc, NPAD + 16, 0.0)
    A = (ixd0, ab0, eb0, semA, semW0)
    B = (ixd1, ab1, eb1, semB, semW1)

    def start(k, buf):
        ixd, ab, eb, sem, semw = buf
        base = w * EPW + k * C

        @pl.when(_real(k, base))
        def _():
            pltpu.async_copy(dst.at[pl.ds(base, C)], ixd, sem)
            pltpu.async_copy(attn.at[pl.ds(base, C)], ab, sem)

    def step(k, buf):
        ixd, ab, eb, sem, semw = buf
        base = w * EPW + k * C

        @pl.when(base < E)
        def _():
            pltpu.make_async_copy(dst.at[pl.ds(base, C)], ixd, sem).wait()
            pltpu.make_async_copy(attn.at[pl.ds(base, C)], ab, sem).wait()

            @pl.when(jnp.asarray(k) >= 2)
            def _():
                pltpu.make_async_copy(eb, ex_out.at[pl.ds(base, C)],
                                      semw).wait()

            def grp(i, _):
                sl = pl.ds(i * 16, 16)
                dv = ixd[sl]
                m = plsc.load_gather(segmax, [dv])
                eb[sl] = jnp.exp(ab[sl] - m)
                return 0
            lax.fori_loop(0, C // 16, grp, 0)

            def acc(gi, _):
                dv = ixd[pl.ds(gi * 16, 16)]
                exv = eb[pl.ds(gi * 16, 16)]
                for e in range(16):
                    d = dv[e]
                    old = denloc[pl.ds(d, 16)]
                    denloc[pl.ds(d, 16)] = jnp.where(
                        mask0, old + exv[e], old)
                return 0
            lax.fori_loop(0, C // 16, acc, 0)
            pltpu.async_copy(eb, ex_out.at[pl.ds(base, C)], semw)

    start(0, A)

    def pipe(g, _):
        k0 = 2 * g
        start(k0 + 1, B)
        step(k0, A)
        start(k0 + 2, A)
        step(k0 + 1, B)
        return 0
    lax.fori_loop(0, NCHUNK // 2, pipe, 0)
    pltpu.make_async_copy(eb0, ex_out.at[pl.ds(0, C)], semW0).wait()
    pltpu.make_async_copy(eb1, ex_out.at[pl.ds(0, C)], semW1).wait()
    pltpu.sync_copy(denloc.at[pl.ds(0, NPAD)], denpart_out.at[w])


C3 = 64
NCHUNK3 = EPW // C3


@functools.partial(
    pl.kernel, mesh=_mesh, compiler_params=_params,
    out_type=jax.ShapeDtypeStruct((2, NPAD, D), jnp.float32),
    scratch_types=(pltpu.VMEM((NPAD,), jnp.float32),
                   pltpu.VMEM((MS,), jnp.float32),
                   pltpu.VMEM((MS,), jnp.float32),
                   pltpu.VMEM((C3,), jnp.int32),
                   pltpu.VMEM((C3,), jnp.int32),
                   pltpu.VMEM((C3,), jnp.int32),
                   pltpu.VMEM((C3,), jnp.int32),
                   pltpu.VMEM((C3,), jnp.float32),
                   pltpu.VMEM((C3,), jnp.float32),
                   pltpu.VMEM((C3,), jnp.float32),
                   pltpu.VMEM((C3,), jnp.float32),
                   pltpu.VMEM((C3, D), jnp.float32),
                   pltpu.VMEM((C3, D), jnp.float32),
                   pltpu.VMEM_SHARED((NPAD, D), jnp.float32),
                   pltpu.SemaphoreType.DMA,
                   pltpu.SemaphoreType.DMA,
                   pltpu.SemaphoreType.DMA,
                   pltpu.SemaphoreType.DMA),
)
def _s3(nh, src, dst, ex, denpart, nzpart_out,
        denom, tmp0, tmp1, ixs0, ixs1, ixd0, ixd1, eb0, eb1, a0, a1,
        rows0, rows1, nzacc, semA, semB, semS0, semS1):
    w = _wid()
    c = lax.axis_index("c")
    s = lax.axis_index("s")
    _fill(denom, NPAD, 0.0)
    _merge(denpart, denom, tmp0, tmp1, semA, jnp.add)

    def zrow(i, _):
        for j in range(8):
            rows0[i, pl.ds(j * 16, 16)] = jnp.zeros((16,), jnp.float32)
        return 0
    lax.fori_loop(0, C3, zrow, 0)
    for m in range(10):
        pltpu.sync_copy(rows0, nzacc.at[pl.ds(s * 640 + m * C3, C3)])
    plsc.subcore_barrier()

    A = (ixs0, ixd0, eb0, a0, rows0, semA, semS0)
    B = (ixs1, ixd1, eb1, a1, rows1, semB, semS1)

    def start(k, buf):
        ixs, ixd, eb, ab, rows, sem, sems = buf
        base = w * EPW + k * C3

        @pl.when(jnp.logical_and(jnp.asarray(k) < NCHUNK3, base < E))
        def _():
            @pl.when(jnp.asarray(k) >= 2)
            def _():
                pltpu.make_async_copy(rows, nzacc.at[ixd], sems).wait()
            pltpu.sync_copy(src.at[pl.ds(base, C3)], ixs)
            pltpu.sync_copy(dst.at[pl.ds(base, C3)], ixd)
            pltpu.sync_copy(ex.at[pl.ds(base, C3)], eb)
            pltpu.async_copy(nh.at[ixs], rows, sem)

    def step(k, buf):
        ixs, ixd, eb, ab, rows, sem, sems = buf
        base = w * EPW + k * C3

        @pl.when(base < E)
        def _():
            pltpu.make_async_copy(nh.at[ixs], rows, sem).wait()

            def grp(i, _):
                sl = pl.ds(i * 16, 16)
                dv = ixd[sl]
                den = plsc.load_gather(denom, [dv])
                ab[sl] = eb[sl] / den
                return 0
            lax.fori_loop(0, C3 // 16, grp, 0)

            def scale(gi, _):
                av16 = ab[pl.ds(gi * 16, 16)]
                for e in range(16):
                    i = gi * 16 + e
                    av = lax.broadcast_in_dim(av16[e], (16,), ())
                    for j in range(8):
                        sl = pl.ds(j * 16, 16)
                        rows[i, sl] = rows[i, sl] * av
                return 0
            lax.fori_loop(0, C3 // 16, scale, 0)
            pltpu.async_copy(rows, nzacc.at[ixd], sems, add=True)

    start(0, A)

    def pipe(g, _):
        k0 = 2 * g
        start(k0 + 1, B)
        step(k0, A)
        start(k0 + 2, A)
        step(k0 + 1, B)
        return 0
    lax.fori_loop(0, NCHUNK3 // 2, pipe, 0)
    pltpu.make_async_copy(rows0, nzacc.at[ixd0], semS0).wait()
    pltpu.make_async_copy(rows1, nzacc.at[ixd1], semS1).wait()
    plsc.subcore_barrier()
    for m in range(10):
        sl = pl.ds(s * 640 + m * C3, C3)
        pltpu.sync_copy(nzacc.at[sl], nzpart_out.at[c, sl])


@functools.partial(
    pl.kernel, mesh=_mesh, compiler_params=_params,
    out_type=jax.ShapeDtypeStruct((E, D), jnp.float32),
    scratch_types=(pltpu.VMEM((C,), jnp.int32),
                   pltpu.VMEM((C,), jnp.int32),
                   pltpu.VMEM((C,), jnp.int32),
                   pltpu.VMEM((C,), jnp.int32),
                   pltpu.VMEM((C, D), jnp.float32),
                   pltpu.VMEM((C, D), jnp.float32),
                   pltpu.VMEM((C, D), jnp.float32),
                   pltpu.VMEM((C, D), jnp.float32),
                   pltpu.SemaphoreType.DMA,
                   pltpu.SemaphoreType.DMA,
                   pltpu.SemaphoreType.DMA,
                   pltpu.SemaphoreType.DMA),
)
def _s4(nz, src, dst, g_out,
        ixs0, ixs1, ixd0, ixd1, ar0, ar1, br0, br1,
        semA, semB, semW0, semW1):
    w = _wid()
    A = (ixs0, ixd0, ar0, br0, semA, semW0)
    B = (ixs1, ixd1, ar1, br1, semB, semW1)

    def start(k, buf):
        ixs, ixd, ar, br, sem, semw = buf
        base = w * EPW + k * C

        @pl.when(_real(k, base))
        def _():
            @pl.when(jnp.asarray(k) >= 2)
            def _():
                pltpu.make_async_copy(ar, g_out.at[pl.ds(base, C)],
                                      semw).wait()
            pltpu.sync_copy(src.at[pl.ds(base, C)], ixs)
            pltpu.sync_copy(dst.at[pl.ds(base, C)], ixd)
            pltpu.async_copy(nz.at[ixs], ar, sem)
            pltpu.async_copy(nz.at[ixd], br, sem)

    def step(k, buf):
        ixs, ixd, ar, br, sem, semw = buf
        base = w * EPW + k * C

        @pl.when(base < E)
        def _():
            pltpu.make_async_copy(nz.at[ixs], ar, sem).wait()
            pltpu.make_async_copy(nz.at[ixd], br, sem).wait()

            def sub(i, _):
                for j in range(8):
                    sl = pl.ds(j * 16, 16)
                    ar[i, sl] = ar[i, sl] - br[i, sl]
                return 0
            lax.fori_loop(0, C, sub, 0)
            pltpu.async_copy(ar, g_out.at[pl.ds(base, C)], semw)

    start(0, A)

    def pipe(g, _):
        k0 = 2 * g
        start(k0 + 1, B)
        step(k0, A)
        start(k0 + 2, A)
        step(k0 + 1, B)
        return 0
    lax.fori_loop(0, NCHUNK // 2, pipe, 0)
    pltpu.make_async_copy(ar0, g_out.at[pl.ds(0, C)], semW0).wait()
    pltpu.make_async_copy(ar1, g_out.at[pl.ds(0, C)], semW1).wait()


def _t0_body(p0, p1, nz_out):
    nz_out[...] = p0[0:N, :] + p1[0:N, :]


def _stats_body(a, b, eps, w1, b1, w2, b2, y_out, s_out, q_out):
    x = (1.0 + eps[...]) * a[...] + b[...]
    h = jnp.maximum(jnp.dot(x, w1[...], preferred_element_type=jnp.float32)
                    + b1[...], 0.0)
    y = jnp.dot(h, w2[...], preferred_element_type=jnp.float32) + b2[...]
    y_out[...] = y

    @pl.when(pl.program_id(0) == 0)
    def _():
        s_out[...] = jnp.zeros_like(s_out)
        q_out[...] = jnp.zeros_like(q_out)
    s_out[...] += jnp.sum(y, axis=0, keepdims=True)
    q_out[...] += jnp.sum(y * y, axis=0, keepdims=True)


def _norm_body(n_rows, y, s, q, gamma, beta, out):
    mu = s[...] / n_rows
    var = q[...] / n_rows - mu * mu
    inv = lax.rsqrt(var + 1e-5)
    out[...] = (y[...] - mu) * inv * gamma[...] + beta[...]


def _mlp_bn(x_a, x_b, eps, w1, b1, w2, b2, gamma, beta, n_rows, blk):
    nblk = n_rows // blk
    row_spec = pl.BlockSpec((blk, D), lambda i: (i, 0))
    vec_spec = pl.BlockSpec((1, D), lambda i: (0, 0))
    mat_spec = pl.BlockSpec((D, D), lambda i: (0, 0))
    y, ssum, qsum = pl.pallas_call(
        _stats_body,
        grid=(nblk,),
        in_specs=[row_spec, row_spec, vec_spec, mat_spec, vec_spec,
                  mat_spec, vec_spec],
        out_specs=[row_spec, vec_spec, vec_spec],
        out_shape=[jax.ShapeDtypeStruct((n_rows, D), jnp.float32),
                   jax.ShapeDtypeStruct((1, D), jnp.float32),
                   jax.ShapeDtypeStruct((1, D), jnp.float32)],
    )(x_a, x_b, eps, w1, b1, w2, b2)
    out = pl.pallas_call(
        functools.partial(_norm_body, float(n_rows)),
        grid=(nblk,),
        in_specs=[row_spec, vec_spec, vec_spec, vec_spec, vec_spec],
        out_specs=row_spec,
        out_shape=jax.ShapeDtypeStruct((n_rows, D), jnp.float32),
    )(y, ssum, qsum, gamma, beta)
    return out


def kernel(nh, eh, edge_index, nf_W1, nf_b1, nf_W2, nf_b2, nf_eps,
           nf_gamma, nf_beta, ef_W1, ef_b1, ef_W2, ef_b2, ef_eps,
           ef_gamma, ef_beta):
    src = edge_index[0]
    dst = edge_index[1]

    attn, maxpart = _s1(nh, eh, src, dst)
    ex, denpart = _s2(attn, dst, maxpart)
    nzpart = _s3(nh, src, dst, ex, denpart)

    nz = pl.pallas_call(
        _t0_body,
        out_shape=jax.ShapeDtypeStruct((N, D), jnp.float32),
    )(nzpart[0], nzpart[1])

    r = lambda v: v.reshape(1, D)
    n_h = _mlp_bn(nh, nz, r(nf_eps), nf_W1, r(nf_b1), nf_W2, r(nf_b2),
                  r(nf_gamma), r(nf_beta), N, 2000)

    g = _s4(nz, src, dst)
    e_h = _mlp_bn(eh, g, r(ef_eps), ef_W1, r(ef_b1), ef_W2, r(ef_b2),
                  r(ef_gamma), r(ef_beta), E, 2000)
    return (n_h, e_h)


# speedup vs baseline: 4.9930x; 1.0505x over previous
import functools
import jax
import jax.numpy as jnp
from jax import lax
from jax.experimental import pallas as pl
from jax.experimental.pallas import tpu as pltpu
from jax.experimental.pallas import tpu_sc as plsc

N = 10000
E = 320000
D = 128
NPAD = 10240
NW = 32
EPW = 10240
C = 128
NCHUNK = EPW // C
NEG = -3.0e38

_mesh = plsc.VectorSubcoreMesh(core_axis_name="c", subcore_axis_name="s")
_params = pltpu.CompilerParams(needs_layout_passes=False)


def _wid():
    return lax.axis_index("s") * 2 + lax.axis_index("c")


def _fill(ref, n, val):
    def body(i, _):
        for u in range(4):
            ref[pl.ds(i * 64 + u * 16, 16)] = jnp.full((16,), val,
                                                       jnp.float32)
        return 0
    lax.fori_loop(0, n // 64, body, 0)


MS = 2560


def _merge(part, acc, tmp0, tmp1, sem, op):
    nstrip = NPAD // MS
    total = NW * nstrip

    def src_of(t):
        return part.at[t // nstrip, pl.ds((t % nstrip) * MS, MS)]

    def fetch(t, tmp):
        @pl.when(jnp.asarray(t) < total)
        def _():
            pltpu.async_copy(src_of(t), tmp, sem)

    def consume(t, tmp):
        pltpu.make_async_copy(src_of(t), tmp, sem).wait()
        boff = (t % nstrip) * MS

        def mrow(i, _):
            for u in range(4):
                o = i * 64 + u * 16
                sl_a = pl.ds(boff + o, 16)
                acc[sl_a] = op(acc[sl_a], tmp[pl.ds(o, 16)])
            return 0
        lax.fori_loop(0, MS // 64, mrow, 0)

    fetch(0, tmp0)

    def pair(p, _):
        t0 = 2 * p
        fetch(t0 + 1, tmp1)
        consume(t0, tmp0)
        fetch(t0 + 2, tmp0)
        consume(t0 + 1, tmp1)
        return 0
    lax.fori_loop(0, total // 2, pair, 0)


def _real(k, base):
    return jnp.logical_and(jnp.asarray(k) < NCHUNK, base < E)


@functools.partial(
    pl.kernel, mesh=_mesh, compiler_params=_params,
    out_type=(jax.ShapeDtypeStruct((E,), jnp.float32),
              jax.ShapeDtypeStruct((NW, NPAD), jnp.float32)),
    scratch_types=(pltpu.VMEM((C,), jnp.int32),
                   pltpu.VMEM((C,), jnp.int32),
                   pltpu.VMEM((C,), jnp.int32),
                   pltpu.VMEM((C,), jnp.int32),
                   pltpu.VMEM((C, D), jnp.float32),
                   pltpu.VMEM((C, D), jnp.float32),
                   pltpu.VMEM((C, D), jnp.float32),
                   pltpu.VMEM((C, D), jnp.float32),
                   pltpu.VMEM((C, D), jnp.float32),
                   pltpu.VMEM((C, D), jnp.float32),
                   pltpu.VMEM((C,), jnp.float32),
                   pltpu.VMEM((C,), jnp.float32),
                   pltpu.VMEM((NPAD + 16,), jnp.float32),
                   pltpu.SemaphoreType.DMA,
                   pltpu.SemaphoreType.DMA,
                   pltpu.SemaphoreType.DMA,
                   pltpu.SemaphoreType.DMA),
)
def _s1(nh, eh, src, dst, attn_out, maxpart_out,
        ixs0, ixs1, ixd0, ixd1, sr0, sr1, dr0, dr1, er0, er1, ab0, ab1,
        maxloc, semA, semB, semW0, semW1):
    w = _wid()
    _fill(maxloc, NPAD + 16, NEG)
    lane = lax.iota(jnp.int32, 16)
    mask0 = lane == 0
    A = (ixs0, ixd0, sr0, dr0, er0, ab0, semA, semW0)
    B = (ixs1, ixd1, sr1, dr1, er1, ab1, semB, semW1)

    def start(k, buf):
        ixs, ixd, sr, dr, er, ab, sem, semw = buf
        base = w * EPW + k * C

        @pl.when(_real(k, base))
        def _():
            pltpu.sync_copy(src.at[pl.ds(base, C)], ixs)
            pltpu.sync_copy(dst.at[pl.ds(base, C)], ixd)
            pltpu.async_copy(nh.at[ixs], sr, sem)
            pltpu.async_copy(nh.at[ixd], dr, sem)
            pltpu.async_copy(eh.at[pl.ds(base, C)], er, sem)

    def step(k, buf):
        ixs, ixd, sr, dr, er, ab, sem, semw = buf
        base = w * EPW + k * C

        @pl.when(base < E)
        def _():
            pltpu.make_async_copy(nh.at[ixs], sr, sem).wait()
            pltpu.make_async_copy(nh.at[ixd], dr, sem).wait()
            pltpu.make_async_copy(eh.at[pl.ds(base, C)], er, sem).wait()

            @pl.when(jnp.asarray(k) >= 2)
            def _():
                pltpu.make_async_copy(ab, attn_out.at[pl.ds(base, C)],
                                      semw).wait()

            def grp(gi, _):
                vec = jnp.zeros((16,), jnp.float32)
                dv = ixd[pl.ds(gi * 16, 16)]
                for e in range(16):
                    i = gi * 16 + e
                    acc = jnp.zeros((16,), jnp.float32)
                    for j in range(8):
                        sl = pl.ds(j * 16, 16)
                        acc += (sr[i, sl] + er[i, sl]) * dr[i, sl]
                    v = jnp.sum(acc)
                    vec = jnp.where(lane == e, v, vec)
                    d = dv[e]
                    old = maxloc[pl.ds(d, 16)]
                    maxloc[pl.ds(d, 16)] = jnp.where(
                        mask0, jnp.maximum(old, v), old)
                ab[pl.ds(gi * 16, 16)] = vec
                return 0
            lax.fori_loop(0, C // 16, grp, 0)
            pltpu.async_copy(ab, attn_out.at[pl.ds(base, C)], semw)

    start(0, A)

    def pipe(g, _):
        k0 = 2 * g
        start(k0 + 1, B)
        step(k0, A)
        start(k0 + 2, A)
        step(k0 + 1, B)
        return 0
    lax.fori_loop(0, NCHUNK // 2, pipe, 0)
    pltpu.make_async_copy(ab0, attn_out.at[pl.ds(0, C)], semW0).wait()
    pltpu.make_async_copy(ab1, attn_out.at[pl.ds(0, C)], semW1).wait()
    pltpu.sync_copy(maxloc.at[pl.ds(0, NPAD)], maxpart_out.at[w])


@functools.partial(
    pl.kernel, mesh=_mesh, compiler_params=_params,
    out_type=(jax.ShapeDtypeStruct((E,), jnp.float32),
              jax.ShapeDtypeStruct((NW, NPAD), jnp.float32)),
    scratch_types=(pltpu.VMEM((NPAD,), jnp.float32),
                   pltpu.VMEM((MS,), jnp.float32),
                   pltpu.VMEM((MS,), jnp.float32),
                   pltpu.VMEM((NPAD + 16,), jnp.float32),
                   pltpu.VMEM((C,), jnp.int32),
                   pltpu.VMEM((C,), jnp.int32),
                   pltpu.VMEM((C,), jnp.float32),
                   pltpu.VMEM((C,), jnp.float32),
                   pltpu.VMEM((C,), jnp.float32),
                   pltpu.VMEM((C,), jnp.float32),
                   pltpu.SemaphoreType.DMA,
                   pltpu.SemaphoreType.DMA,
                   pltpu.SemaphoreType.DMA,
                   pltpu.SemaphoreType.DMA),
)
def _s2(attn, dst, maxpart, ex_out, denpart_out,
        segmax, tmp0, tmp1, denloc, ixd0, ixd1, ab0, ab1, eb0, eb1,
        semA, semB, semW0, semW1):
    w = _wid()
    mask0 = lax.iota(jnp.int32, 16) == 0
    _fill(segmax, NPAD, NEG)
    _merge(maxpart, segmax, tmp0, tmp1, semA, jnp.maximum)

    def fix(i, _):
        for u in range(4):
            sl = pl.ds(i * 64 + u * 16, 16)
            v = segmax[sl]
            segmax[sl] = jnp.where(v < NEG, jnp.zeros((16,), jnp.float32),
                                   v)
        return 0
    lax.fori_loop(0, NPAD // 64, fix, 0)

    _fill(denloc, NPAD + 16, 0.0)
    A = (ixd0, ab0, eb0, semA, semW0)
    B = (ixd1, ab1, eb1, semB, semW1)

    def start(k, buf):
        ixd, ab, eb, sem, semw = buf
        base = w * EPW + k * C

        @pl.when(_real(k, base))
        def _():
            pltpu.async_copy(dst.at[pl.ds(base, C)], ixd, sem)
            pltpu.async_copy(attn.at[pl.ds(base, C)], ab, sem)

    def step(k, buf):
        ixd, ab, eb, sem, semw = buf
        base = w * EPW + k * C

        @pl.when(base < E)
        def _():
            pltpu.make_async_copy(dst.at[pl.ds(base, C)], ixd, sem).wait()
            pltpu.make_async_copy(attn.at[pl.ds(base, C)], ab, sem).wait()

            @pl.when(jnp.asarray(k) >= 2)
            def _():
                pltpu.make_async_copy(eb, ex_out.at[pl.ds(base, C)],
                                      semw).wait()

            def grp(i, _):
                sl = pl.ds(i * 16, 16)
                dv = ixd[sl]
                m = plsc.load_gather(segmax, [dv])
                eb[sl] = jnp.exp(ab[sl] - m)
                return 0
            lax.fori_loop(0, C // 16, grp, 0)

            def acc(gi, _):
                sl = pl.ds(gi * 16, 16)
                plsc.addupdate_scatter(denloc, [ixd[sl]], eb[sl])
                return 0
            lax.fori_loop(0, C // 16, acc, 0)
            pltpu.async_copy(eb, ex_out.at[pl.ds(base, C)], semw)

    start(0, A)

    def pipe(g, _):
        k0 = 2 * g
        start(k0 + 1, B)
        step(k0, A)
        start(k0 + 2, A)
        step(k0 + 1, B)
        return 0
    lax.fori_loop(0, NCHUNK // 2, pipe, 0)
    pltpu.make_async_copy(eb0, ex_out.at[pl.ds(0, C)], semW0).wait()
    pltpu.make_async_copy(eb1, ex_out.at[pl.ds(0, C)], semW1).wait()
    pltpu.sync_copy(denloc.at[pl.ds(0, NPAD)], denpart_out.at[w])


C3 = 64
NCHUNK3 = EPW // C3


@functools.partial(
    pl.kernel, mesh=_mesh, compiler_params=_params,
    out_type=jax.ShapeDtypeStruct((2, NPAD, D), jnp.float32),
    scratch_types=(pltpu.VMEM((NPAD,), jnp.float32),
                   pltpu.VMEM((MS,), jnp.float32),
                   pltpu.VMEM((MS,), jnp.float32),
                   pltpu.VMEM((C3,), jnp.int32),
                   pltpu.VMEM((C3,), jnp.int32),
                   pltpu.VMEM((C3,), jnp.int32),
                   pltpu.VMEM((C3,), jnp.int32),
                   pltpu.VMEM((C3,), jnp.float32),
                   pltpu.VMEM((C3,), jnp.float32),
                   pltpu.VMEM((C3,), jnp.float32),
                   pltpu.VMEM((C3,), jnp.float32),
                   pltpu.VMEM((C3, D), jnp.float32),
                   pltpu.VMEM((C3, D), jnp.float32),
                   pltpu.VMEM_SHARED((NPAD, D), jnp.float32),
                   pltpu.SemaphoreType.DMA,
                   pltpu.SemaphoreType.DMA,
                   pltpu.SemaphoreType.DMA,
                   pltpu.SemaphoreType.DMA),
)
def _s3(nh, src, dst, ex, denpart, nzpart_out,
        denom, tmp0, tmp1, ixs0, ixs1, ixd0, ixd1, eb0, eb1, a0, a1,
        rows0, rows1, nzacc, semA, semB, semS0, semS1):
    w = _wid()
    c = lax.axis_index("c")
    s = lax.axis_index("s")
    _fill(denom, NPAD, 0.0)
    _merge(denpart, denom, tmp0, tmp1, semA, jnp.add)

    def zrow(i, _):
        for j in range(8):
            rows0[i, pl.ds(j * 16, 16)] = jnp.zeros((16,), jnp.float32)
        return 0
    lax.fori_loop(0, C3, zrow, 0)
    for m in range(10):
        pltpu.sync_copy(rows0, nzacc.at[pl.ds(s * 640 + m * C3, C3)])
    plsc.subcore_barrier()

    A = (ixs0, ixd0, eb0, a0, rows0, semA, semS0)
    B = (ixs1, ixd1, eb1, a1, rows1, semB, semS1)

    def start(k, buf):
        ixs, ixd, eb, ab, rows, sem, sems = buf
        base = w * EPW + k * C3

        @pl.when(jnp.logical_and(jnp.asarray(k) < NCHUNK3, base < E))
        def _():
            @pl.when(jnp.asarray(k) >= 2)
            def _():
                pltpu.make_async_copy(rows, nzacc.at[ixd], sems).wait()
            pltpu.sync_copy(src.at[pl.ds(base, C3)], ixs)
            pltpu.sync_copy(dst.at[pl.ds(base, C3)], ixd)
            pltpu.sync_copy(ex.at[pl.ds(base, C3)], eb)
            pltpu.async_copy(nh.at[ixs], rows, sem)

    def step(k, buf):
        ixs, ixd, eb, ab, rows, sem, sems = buf
        base = w * EPW + k * C3

        @pl.when(base < E)
        def _():
            pltpu.make_async_copy(nh.at[ixs], rows, sem).wait()

            def grp(i, _):
                sl = pl.ds(i * 16, 16)
                dv = ixd[sl]
                den = plsc.load_gather(denom, [dv])
                ab[sl] = eb[sl] / den
                return 0
            lax.fori_loop(0, C3 // 16, grp, 0)

            def scale(gi, _):
                av16 = ab[pl.ds(gi * 16, 16)]
                for e in range(16):
                    i = gi * 16 + e
                    av = lax.broadcast_in_dim(av16[e], (16,), ())
                    for j in range(8):
                        sl = pl.ds(j * 16, 16)
                        rows[i, sl] = rows[i, sl] * av
                return 0
            lax.fori_loop(0, C3 // 16, scale, 0)
            pltpu.async_copy(rows, nzacc.at[ixd], sems, add=True)

    start(0, A)

    def pipe(g, _):
        k0 = 2 * g
        start(k0 + 1, B)
        step(k0, A)
        start(k0 + 2, A)
        step(k0 + 1, B)
        return 0
    lax.fori_loop(0, NCHUNK3 // 2, pipe, 0)
    pltpu.make_async_copy(rows0, nzacc.at[ixd0], semS0).wait()
    pltpu.make_async_copy(rows1, nzacc.at[ixd1], semS1).wait()
    plsc.subcore_barrier()
    for m in range(10):
        sl = pl.ds(s * 640 + m * C3, C3)
        pltpu.sync_copy(nzacc.at[sl], nzpart_out.at[c, sl])


@functools.partial(
    pl.kernel, mesh=_mesh, compiler_params=_params,
    out_type=jax.ShapeDtypeStruct((E, D), jnp.float32),
    scratch_types=(pltpu.VMEM((C,), jnp.int32),
                   pltpu.VMEM((C,), jnp.int32),
                   pltpu.VMEM((C,), jnp.int32),
                   pltpu.VMEM((C,), jnp.int32),
                   pltpu.VMEM((C, D), jnp.float32),
                   pltpu.VMEM((C, D), jnp.float32),
                   pltpu.VMEM((C, D), jnp.float32),
                   pltpu.VMEM((C, D), jnp.float32),
                   pltpu.SemaphoreType.DMA,
                   pltpu.SemaphoreType.DMA,
                   pltpu.SemaphoreType.DMA,
                   pltpu.SemaphoreType.DMA),
)
def _s4(nz, src, dst, g_out,
        ixs0, ixs1, ixd0, ixd1, ar0, ar1, br0, br1,
        semA, semB, semW0, semW1):
    w = _wid()
    A = (ixs0, ixd0, ar0, br0, semA, semW0)
    B = (ixs1, ixd1, ar1, br1, semB, semW1)

    def start(k, buf):
        ixs, ixd, ar, br, sem, semw = buf
        base = w * EPW + k * C

        @pl.when(_real(k, base))
        def _():
            @pl.when(jnp.asarray(k) >= 2)
            def _():
                pltpu.make_async_copy(ar, g_out.at[pl.ds(base, C)],
                                      semw).wait()
            pltpu.sync_copy(src.at[pl.ds(base, C)], ixs)
            pltpu.sync_copy(dst.at[pl.ds(base, C)], ixd)
            pltpu.async_copy(nz.at[ixs], ar, sem)
            pltpu.async_copy(nz.at[ixd], br, sem)

    def step(k, buf):
        ixs, ixd, ar, br, sem, semw = buf
        base = w * EPW + k * C

        @pl.when(base < E)
        def _():
            pltpu.make_async_copy(nz.at[ixs], ar, sem).wait()
            pltpu.make_async_copy(nz.at[ixd], br, sem).wait()

            def sub(i, _):
                for j in range(8):
                    sl = pl.ds(j * 16, 16)
                    ar[i, sl] = ar[i, sl] - br[i, sl]
                return 0
            lax.fori_loop(0, C, sub, 0)
            pltpu.async_copy(ar, g_out.at[pl.ds(base, C)], semw)

    start(0, A)

    def pipe(g, _):
        k0 = 2 * g
        start(k0 + 1, B)
        step(k0, A)
        start(k0 + 2, A)
        step(k0 + 1, B)
        return 0
    lax.fori_loop(0, NCHUNK // 2, pipe, 0)
    pltpu.make_async_copy(ar0, g_out.at[pl.ds(0, C)], semW0).wait()
    pltpu.make_async_copy(ar1, g_out.at[pl.ds(0, C)], semW1).wait()


def _t0_body(p0, p1, nz_out):
    nz_out[...] = p0[0:N, :] + p1[0:N, :]


def _stats_body(a, b, eps, w1, b1, w2, b2, y_out, s_out, q_out):
    x = (1.0 + eps[...]) * a[...] + b[...]
    h = jnp.maximum(jnp.dot(x, w1[...], preferred_element_type=jnp.float32)
                    + b1[...], 0.0)
    y = jnp.dot(h, w2[...], preferred_element_type=jnp.float32) + b2[...]
    y_out[...] = y.astype(jnp.bfloat16)

    @pl.when(pl.program_id(0) == 0)
    def _():
        s_out[...] = jnp.zeros_like(s_out)
        q_out[...] = jnp.zeros_like(q_out)
    s_out[...] += jnp.sum(y, axis=0, keepdims=True)
    q_out[...] += jnp.sum(y * y, axis=0, keepdims=True)


def _norm_body(n_rows, y, s, q, gamma, beta, out):
    mu = s[...] / n_rows
    var = q[...] / n_rows - mu * mu
    inv = lax.rsqrt(var + 1e-5)
    out[...] = ((y[...].astype(jnp.float32) - mu) * inv * gamma[...]
                + beta[...])


def _mlp_bn(x_a, x_b, eps, w1, b1, w2, b2, gamma, beta, n_rows, blk):
    nblk = n_rows // blk
    row_spec = pl.BlockSpec((blk, D), lambda i: (i, 0))
    vec_spec = pl.BlockSpec((1, D), lambda i: (0, 0))
    mat_spec = pl.BlockSpec((D, D), lambda i: (0, 0))
    y, ssum, qsum = pl.pallas_call(
        _stats_body,
        grid=(nblk,),
        in_specs=[row_spec, row_spec, vec_spec, mat_spec, vec_spec,
                  mat_spec, vec_spec],
        out_specs=[row_spec, vec_spec, vec_spec],
        out_shape=[jax.ShapeDtypeStruct((n_rows, D), jnp.bfloat16),
                   jax.ShapeDtypeStruct((1, D), jnp.float32),
                   jax.ShapeDtypeStruct((1, D), jnp.float32)],
    )(x_a, x_b, eps, w1, b1, w2, b2)
    out = pl.pallas_call(
        functools.partial(_norm_body, float(n_rows)),
        grid=(nblk,),
        in_specs=[row_spec, vec_spec, vec_spec, vec_spec, vec_spec],
        out_specs=row_spec,
        out_shape=jax.ShapeDtypeStruct((n_rows, D), jnp.float32),
    )(y, ssum, qsum, gamma, beta)
    return out


def kernel(nh, eh, edge_index, nf_W1, nf_b1, nf_W2, nf_b2, nf_eps,
           nf_gamma, nf_beta, ef_W1, ef_b1, ef_W2, ef_b2, ef_eps,
           ef_gamma, ef_beta):
    src = edge_index[0]
    dst = edge_index[1]

    attn, maxpart = _s1(nh, eh, src, dst)
    ex, denpart = _s2(attn, dst, maxpart)
    nzpart = _s3(nh, src, dst, ex, denpart)

    nz = pl.pallas_call(
        _t0_body,
        out_shape=jax.ShapeDtypeStruct((N, D), jnp.float32),
    )(nzpart[0], nzpart[1])

    r = lambda v: v.reshape(1, D)
    n_h = _mlp_bn(nh, nz, r(nf_eps), nf_W1, r(nf_b1), nf_W2, r(nf_b2),
                  r(nf_gamma), r(nf_beta), N, 2000)

    g = _s4(nz, src, dst)
    e_h = _mlp_bn(eh, g, r(ef_eps), ef_W1, r(ef_b1), ef_W2, r(ef_b2),
                  r(ef_gamma), r(ef_beta), E, 2000)
    return (n_h, e_h)
